# baseline (speedup 1.0000x reference)
import jax
import jax.numpy as jnp
from jax.experimental import pallas as pl
from jax.experimental.pallas import tpu as pltpu

_LEVELS = 6
_R = _LEVELS - 1


def _fused_kernel(w_hbm, x_hbm, W1_hbm, r1_hbm, b1_hbm, W2_hbm, r2_hbm,
                  b2_hbm, out_ref, w_s, x_s, W1_s, r1_s, b1_s, W2_s, r2_s,
                  b2_s, sems):
    cp = []
    srcs = [w_hbm, x_hbm, W1_hbm.at[1:_LEVELS], r1_hbm, b1_hbm,
            W2_hbm.at[1:_LEVELS], r2_hbm, b2_hbm]
    dsts = [w_s, x_s, W1_s, r1_s, b1_s, W2_s, r2_s, b2_s]
    for i, (src, dst) in enumerate(zip(srcs, dsts)):
        c = pltpu.make_async_copy(src, dst, sems.at[i])
        c.start()
        cp.append(c)

    cp[0].wait()
    wt = w_s[...].T
    n_med = wt.shape[0]

    cmasks = []
    csums = []
    for r in range(1, _LEVELS):
        c = (wt > r / _LEVELS).astype(jnp.float32)
        csums.append(jnp.sum(c, axis=1, keepdims=True))
        cmasks.append(c.astype(jnp.bfloat16))
    inv_cnts = []
    for k in range(_R):
        s_hi = csums[k + 1] if k + 1 < _R else 0.0
        inv_cnts.append(1.0 / jnp.maximum(csums[k] - s_hi, 1.0))

    cp[1].wait()
    xb1 = x_s[...].astype(jnp.bfloat16)
    cp[3].wait()
    cp[4].wait()
    rt1 = jnp.dot(xb1, r1_s[...].astype(jnp.bfloat16),
                  preferred_element_type=jnp.float32) + b1_s[...]

    def layer(xb, W_s_ref, rt):
        ps = [jnp.dot(c, xb, preferred_element_type=jnp.float32)
              for c in cmasks]
        agg = jnp.zeros((n_med, xb.shape[1]), dtype=jnp.float32)
        for k in range(_R):
            p_hi = ps[k + 1] if k + 1 < _R else 0.0
            g = (ps[k] - p_hi) * inv_cnts[k]
            agg = agg + jnp.dot(g.astype(jnp.bfloat16),
                                W_s_ref[k].astype(jnp.bfloat16),
                                preferred_element_type=jnp.float32)
        return jnp.concatenate([rt[:n_med, :] + agg, rt[n_med:, :]], axis=0)

    cp[2].wait()
    h1 = jnp.maximum(layer(xb1, W1_s, rt1), 0.0)

    xb2 = h1.astype(jnp.bfloat16)
    cp[6].wait()
    cp[7].wait()
    rt2 = jnp.dot(xb2, r2_s[...].astype(jnp.bfloat16),
                  preferred_element_type=jnp.float32) + b2_s[...]
    cp[5].wait()
    out_ref[...] = layer(xb2, W2_s, rt2)


@jax.jit
def kernel(emb_entity, emb_mole, entity_mole_weights, W1, root1, b1, W2,
           root2, b2):
    del emb_mole
    n_ent, d = emb_entity.shape[1], emb_entity.shape[2]
    n_med = entity_mole_weights.shape[1]
    x = emb_entity.reshape(n_ent, d)

    any_spec = pl.BlockSpec(memory_space=pl.ANY)
    out = pl.pallas_call(
        _fused_kernel,
        out_shape=jax.ShapeDtypeStruct((n_ent, d), jnp.float32),
        in_specs=[any_spec] * 8,
        scratch_shapes=[
            pltpu.VMEM((n_ent, n_med), jnp.float32),
            pltpu.VMEM((n_ent, d), jnp.float32),
            pltpu.VMEM((_R, d, d), jnp.float32),
            pltpu.VMEM((d, d), jnp.float32),
            pltpu.VMEM((1, d), jnp.float32),
            pltpu.VMEM((_R, d, d), jnp.float32),
            pltpu.VMEM((d, d), jnp.float32),
            pltpu.VMEM((1, d), jnp.float32),
            pltpu.SemaphoreType.DMA((8,)),
        ],
    )(entity_mole_weights, x, W1, root1, b1.reshape(1, d), W2, root2,
      b2.reshape(1, d))
    return out


# speedup vs baseline: 1.1023x; 1.1023x over previous
import jax
import jax.numpy as jnp
from jax.experimental import pallas as pl
from jax.experimental.pallas import tpu as pltpu

_LEVELS = 6


def _fused_kernel(w_ref, x_ref, W1_ref, r1_ref, b1_ref, W2_ref, r2_ref,
                  b2_ref, out_ref):
    n_med = w_ref.shape[1]

    xb1 = x_ref[...].astype(jnp.bfloat16)
    rt1 = jnp.dot(xb1, r1_ref[...].astype(jnp.bfloat16),
                  preferred_element_type=jnp.float32) + b1_ref[...]

    w = w_ref[...]

    cmasks = []
    csums = []
    for r in range(1, _LEVELS):
        c = (w > r / _LEVELS).astype(jnp.float32)
        csums.append(jnp.sum(c, axis=0, keepdims=True))
        cmasks.append(c.astype(jnp.bfloat16))
    cs = jnp.concatenate(csums, axis=0).T
    inv_cnts = []
    for k in range(_LEVELS - 1):
        s_hi = cs[:, k + 1:k + 2] if k + 1 < _LEVELS - 1 else 0.0
        inv_cnts.append(1.0 / jnp.maximum(cs[:, k:k + 1] - s_hi, 1.0))

    _dn = (((0,), (0,)), ((), ()))
    def layer(xb, W_ref, rt):
        ps = [jax.lax.dot_general(c, xb, _dn,
                                  preferred_element_type=jnp.float32)
              for c in cmasks]
        agg = jnp.zeros((n_med, xb.shape[1]), dtype=jnp.float32)
        for k in range(_LEVELS - 1):
            p_hi = ps[k + 1] if k + 1 < _LEVELS - 1 else 0.0
            g = (ps[k] - p_hi) * inv_cnts[k]
            agg = agg + jnp.dot(g.astype(jnp.bfloat16),
                                W_ref[k + 1].astype(jnp.bfloat16),
                                preferred_element_type=jnp.float32)
        return jnp.concatenate([rt[:n_med, :] + agg, rt[n_med:, :]], axis=0)

    h1 = jnp.maximum(layer(xb1, W1_ref, rt1), 0.0)
    xb2 = h1.astype(jnp.bfloat16)
    rt2 = jnp.dot(xb2, r2_ref[...].astype(jnp.bfloat16),
                  preferred_element_type=jnp.float32) + b2_ref[...]
    out_ref[...] = layer(xb2, W2_ref, rt2)


@jax.jit
def kernel(emb_entity, emb_mole, entity_mole_weights, W1, root1, b1, W2,
           root2, b2):
    del emb_mole
    n_ent, d = emb_entity.shape[1], emb_entity.shape[2]
    x = emb_entity.reshape(n_ent, d)

    out = pl.pallas_call(
        _fused_kernel,
        out_shape=jax.ShapeDtypeStruct((n_ent, d), jnp.float32),
    )(entity_mole_weights, x, W1, root1, b1.reshape(1, d), W2, root2,
      b2.reshape(1, d))
    return out


# speedup vs baseline: 1.1029x; 1.0005x over previous
import jax
import jax.numpy as jnp
from jax.experimental import pallas as pl
from jax.experimental.pallas import tpu as pltpu

_LEVELS = 6


def _fused_kernel(w_ref, x_ref, W1_ref, r1_ref, b1_ref, W2_ref, r2_ref,
                  b2_ref, out_ref):
    n_med = w_ref.shape[1]

    xb1 = x_ref[...].astype(jnp.bfloat16)
    rt1 = jnp.dot(xb1, r1_ref[...].astype(jnp.bfloat16),
                  preferred_element_type=jnp.float32) + b1_ref[...]

    w = w_ref[...]

    cmasks = []
    csums = []
    for r in range(1, _LEVELS):
        c = (w > r / _LEVELS).astype(jnp.float32)
        csums.append(jnp.sum(c, axis=0, keepdims=True))
        cmasks.append(c.astype(jnp.bfloat16))
    cs = jnp.concatenate(csums, axis=0).T
    inv_cnts = []
    for k in range(_LEVELS - 1):
        s_hi = cs[:, k + 1:k + 2] if k + 1 < _LEVELS - 1 else 0.0
        inv_cnts.append(1.0 / jnp.maximum(cs[:, k:k + 1] - s_hi, 1.0))

    _dn = (((0,), (0,)), ((), ()))

    def agg_of(ps, W_ref):
        agg = jnp.zeros((n_med, ps[0].shape[1]), dtype=jnp.float32)
        for k in range(_LEVELS - 1):
            p_hi = ps[k + 1] if k + 1 < _LEVELS - 1 else 0.0
            g = (ps[k] - p_hi) * inv_cnts[k]
            agg = agg + jnp.dot(g.astype(jnp.bfloat16),
                                W_ref[k + 1].astype(jnp.bfloat16),
                                preferred_element_type=jnp.float32)
        return agg

    ps1 = [jax.lax.dot_general(c, xb1, _dn,
                               preferred_element_type=jnp.float32)
           for c in cmasks]
    agg1 = agg_of(ps1, W1_ref)

    rb2 = r2_ref[...].astype(jnp.bfloat16)
    bb2 = jnp.maximum(rt1[n_med:, :], 0.0).astype(jnp.bfloat16)
    out_ref[n_med:, :] = jnp.dot(bb2, rb2,
                                 preferred_element_type=jnp.float32) \
        + b2_ref[...]
    tb2 = jnp.maximum(rt1[:n_med, :] + agg1, 0.0).astype(jnp.bfloat16)
    rt2_t = jnp.dot(tb2, rb2, preferred_element_type=jnp.float32) \
        + b2_ref[...]
    ps2 = [jax.lax.dot_general(c[n_med:, :], bb2, _dn,
                               preferred_element_type=jnp.float32)
           + jax.lax.dot_general(c[:n_med, :], tb2, _dn,
                                 preferred_element_type=jnp.float32)
           for c in cmasks]
    out_ref[:n_med, :] = rt2_t + agg_of(ps2, W2_ref)


@jax.jit
def kernel(emb_entity, emb_mole, entity_mole_weights, W1, root1, b1, W2,
           root2, b2):
    del emb_mole
    n_ent, d = emb_entity.shape[1], emb_entity.shape[2]
    x = emb_entity.reshape(n_ent, d)

    out = pl.pallas_call(
        _fused_kernel,
        out_shape=jax.ShapeDtypeStruct((n_ent, d), jnp.float32),
    )(entity_mole_weights, x, W1, root1, b1.reshape(1, d), W2, root2,
      b2.reshape(1, d))
    return out
